# baseline (speedup 1.0000x reference)
import functools

import jax
import jax.numpy as jnp
from jax import lax
from jax.experimental import pallas as pl
from jax.experimental.pallas import tpu as pltpu
from jax.experimental.pallas import tpu_sc as plsc

B = 16384
L = 200
C = 50
CP = 64
M = B * 202
NC, NS = 2, 16
NW = NC * NS
RPT = B // NW
SBR = 32
NSB = RPT // SBR
HIST = 1024
PB = B // 2
PPT = PB // NW

_mesh = plsc.VectorSubcoreMesh(core_axis_name="c", subcore_axis_name="s")
_sc_params = pltpu.CompilerParams(needs_layout_passes=False)


@functools.partial(
    pl.kernel,
    out_type=[
        jax.ShapeDtypeStruct((B,), jnp.int32),
        jax.ShapeDtypeStruct((NW, 16), jnp.int32),
    ],
    mesh=_mesh,
    compiler_params=_sc_params,
    scratch_types=[
        pltpu.VMEM((SBR, L), jnp.int32),
        pltpu.VMEM((HIST,), jnp.int32),
        pltpu.VMEM((HIST,), jnp.int32),
        pltpu.VMEM((RPT,), jnp.int32),
        pltpu.VMEM((16,), jnp.int32),
        pltpu.SemaphoreType.DMA,
    ],
)
def _sc_unique_counts(lab_hbm, inc_hbm, tot_hbm, lab_buf, hist_a, hist_b,
                      inc_buf, tot_buf, sem):
    wid = lax.axis_index("s") * NC + lax.axis_index("c")
    base = wid * RPT
    lanes = lax.iota(jnp.int32, 16)
    zeros = jnp.zeros((16,), jnp.int32)
    zerosf = jnp.zeros((16,), jnp.float32)
    ones = jnp.ones((16,), jnp.int32)
    lane0 = lanes == 0
    hi8 = lanes >= 8

    for i in range(HIST // 16):
        hist_a[pl.ds(16 * i, 16)] = zeros
        hist_b[pl.ds(16 * i, 16)] = zeros

    def sb_body(sb, _):
        pltpu.sync_copy(lab_hbm.at[pl.ds(base + sb * SBR, SBR)], lab_buf)

        def one_row(r, hist):
            labs = [lab_buf[r, pl.ds(16 * k, 16)] for k in range(12)]
            tail = lab_buf[r, pl.ds(L - 16, 16)]
            for lv in labs:
                plsc.addupdate_scatter(hist, [lv], ones)
            plsc.addupdate_scatter(hist, [tail], ones, mask=hi8)
            return labs, tail

        def finish_row(r, hist, labs, tail):
            acc = jnp.zeros((16,), jnp.float32)
            for lv in labs:
                occ = plsc.load_gather(hist, [lv])
                acc = acc + 1.0 / occ.astype(jnp.float32)
            tocc = plsc.load_gather(hist, [tail])
            acc = acc + jnp.where(
                hi8, 1.0 / tocc.astype(jnp.float32), zerosf)
            s = jnp.sum(acc)
            inc = (s + 0.25).astype(jnp.int32) + 1
            ridx = jnp.full((16,), sb * SBR + r, jnp.int32)
            plsc.store_scatter(inc_buf, [ridx],
                               jnp.full((16,), inc, jnp.int32), mask=lane0)
            for lv in labs:
                plsc.store_scatter(hist, [lv], zeros)
            plsc.store_scatter(hist, [tail], zeros)

        def row_pair(r2, _):
            ra, rb = 2 * r2, 2 * r2 + 1
            la, ta = one_row(ra, hist_a)
            lb, tb = one_row(rb, hist_b)
            finish_row(ra, hist_a, la, ta)
            finish_row(rb, hist_b, lb, tb)
            return 0

        lax.fori_loop(0, SBR // 2, row_pair, 0)
        return 0

    lax.fori_loop(0, NSB, sb_body, 0)

    tacc = zeros
    for j in range(RPT // 16):
        tacc = tacc + inc_buf[pl.ds(16 * j, 16)]
    total = jnp.sum(tacc)
    tot_buf[...] = jnp.full((16,), total, jnp.int32)
    pltpu.sync_copy(inc_buf, inc_hbm.at[pl.ds(base, RPT)])
    pltpu.sync_copy(tot_buf, tot_hbm.at[wid])


@functools.partial(
    pl.kernel,
    out_type=jax.ShapeDtypeStruct((PB, 128), jnp.float32),
    mesh=_mesh,
    compiler_params=_sc_params,
    scratch_types=[
        pltpu.VMEM((RPT,), jnp.int32),
        pltpu.VMEM((NW, 16), jnp.int32),
        pltpu.VMEM((RPT,), jnp.int32),
        pltpu.VMEM((16, 128), jnp.int32),
        pltpu.VMEM((16, 128), jnp.float32),
        pltpu.VMEM((16, 128), jnp.float32),
        pltpu.SemaphoreType.DMA,
    ],
)
def _sc_gather(inp_hbm, inc_hbm, tot_hbm, cl_hbm, out_hbm, inc_buf, tots,
               offs_buf, cl_buf, win_buf, val_buf, sem):
    wid = lax.axis_index("s") * NC + lax.axis_index("c")
    base = wid * RPT
    pbase = wid * PPT
    pltpu.sync_copy(inc_hbm.at[pl.ds(base, RPT)], inc_buf)
    pltpu.sync_copy(tot_hbm, tots)

    acc = jnp.zeros((16,), jnp.int32)
    for w in range(NW):
        row = tots[w, pl.ds(0, 16)]
        acc = acc + jnp.where(w < wid, row, 0)
    gbase = jnp.max(acc)

    def cums(j, carry):
        v = inc_buf[pl.ds(16 * j, 16)]
        c = plsc.cumsum(v)
        offs_buf[pl.ds(16 * j, 16)] = c - v + carry
        return carry + jnp.max(c)

    lax.fori_loop(0, RPT // 16, cums, gbase)

    def sb_body(sb, _):
        pltpu.sync_copy(cl_hbm.at[pl.ds(pbase + 16 * sb, 16)], cl_buf)

        def row_win(r, _):
            off_v = plsc.load_gather(
                offs_buf, [jnp.full((16,), sb * SBR + r, jnp.int32)])
            off_s = jnp.max(off_v)
            start = pl.multiple_of(off_s - lax.rem(off_s, 8), 8)
            prow = lax.div(r, 2)
            col0 = pl.multiple_of(lax.rem(r, 2) * 64, 64)
            pltpu.async_copy(inp_hbm.at[pl.ds(start, 64)],
                             win_buf.at[prow, pl.ds(col0, 64)], sem)
            return 0

        lax.fori_loop(0, SBR, row_win, 0)
        pltpu.make_async_copy(
            out_hbm.at[pl.ds(pbase + 16 * sb, 16)], win_buf, sem).wait()

        def row_sel(r, _):
            off_v = plsc.load_gather(
                offs_buf, [jnp.full((16,), sb * SBR + r, jnp.int32)])
            off_lo = lax.rem(off_v, 8)
            prow = lax.div(r, 2)
            col0 = lax.rem(r, 2) * 64
            prow_v = jnp.full((16,), prow, jnp.int32)
            for k in range(CP // 16):
                clk = cl_buf[prow, pl.ds(col0 + 16 * k, 16)]
                v = plsc.load_gather(win_buf, [prow_v, clk + off_lo + col0])
                val_buf[prow, pl.ds(col0 + 16 * k, 16)] = v
            return 0

        lax.fori_loop(0, SBR, row_sel, 0)
        pltpu.sync_copy(val_buf, out_hbm.at[pl.ds(pbase + 16 * sb, 16)])
        return 0

    lax.fori_loop(0, NSB, sb_body, 0)


_TC_GRID = 32
_TC_ROWS = PB // _TC_GRID


def _tc_loss_body(v_ref, out_ref, acc_ref):
    i = pl.program_id(0)

    @pl.when(i == 0)
    def _():
        acc_ref[...] = jnp.zeros((8, 128), jnp.float32)

    x = v_ref[...]
    t = -jnp.log(jax.nn.sigmoid(x) + 1e-08)
    col = lax.broadcasted_iota(jnp.int32, (_TC_ROWS, 128), 1)
    t = jnp.where(lax.rem(col, 64) < C, t, 0.0)
    acc_ref[...] += t.reshape(_TC_ROWS // 8, 8, 128).sum(axis=0)

    @pl.when(i == _TC_GRID - 1)
    def _():
        out_ref[0, 0] = jnp.sum(acc_ref[...]) * (1.0 / (B * C))


_tc_loss = pl.pallas_call(
    _tc_loss_body,
    out_shape=jax.ShapeDtypeStruct((1, 1), jnp.float32),
    grid=(_TC_GRID,),
    in_specs=[pl.BlockSpec((_TC_ROWS, 128), lambda i: (i, 0))],
    out_specs=pl.BlockSpec(memory_space=pltpu.SMEM),
    scratch_shapes=[pltpu.VMEM((8, 128), jnp.float32)],
)


def kernel(input, labels, labels_clicked):
    labels = labels.astype(jnp.int32)
    clicked = labels_clicked.astype(jnp.int32)
    cl_pad = jnp.concatenate(
        [clicked, jnp.zeros((B, CP - C), jnp.int32)], axis=1).reshape(PB, 128)
    inc, tot = _sc_unique_counts(labels)
    vals = _sc_gather(input, inc, tot, cl_pad)
    return _tc_loss(vals)[0, 0]


# speedup vs baseline: 1.1225x; 1.1225x over previous
import functools

import jax
import jax.numpy as jnp
from jax import lax
from jax.experimental import pallas as pl
from jax.experimental.pallas import tpu as pltpu
from jax.experimental.pallas import tpu_sc as plsc

B = 16384
L = 200
C = 50
CP = 64
M = B * 202
NC, NS = 2, 16
NW = NC * NS
RPT = B // NW
SBR = 32
NSB = RPT // SBR
HIST = 1024
PB = B // 2
PPT = PB // NW

_mesh = plsc.VectorSubcoreMesh(core_axis_name="c", subcore_axis_name="s")
_sc_params = pltpu.CompilerParams(needs_layout_passes=False)


@functools.partial(
    pl.kernel,
    out_type=[
        jax.ShapeDtypeStruct((B,), jnp.int32),
        jax.ShapeDtypeStruct((NW, 16), jnp.int32),
    ],
    mesh=_mesh,
    compiler_params=_sc_params,
    scratch_types=[
        pltpu.VMEM((SBR, L), jnp.int32),
        pltpu.VMEM((HIST,), jnp.int32),
        pltpu.VMEM((HIST,), jnp.int32),
        pltpu.VMEM((RPT,), jnp.int32),
        pltpu.VMEM((16,), jnp.int32),
        pltpu.SemaphoreType.DMA,
    ],
)
def _sc_unique_counts(lab_hbm, inc_hbm, tot_hbm, lab_buf, hist_a, hist_b,
                      inc_buf, tot_buf, sem):
    wid = lax.axis_index("s") * NC + lax.axis_index("c")
    base = wid * RPT
    lanes = lax.iota(jnp.int32, 16)
    zeros = jnp.zeros((16,), jnp.int32)
    zerosf = jnp.zeros((16,), jnp.float32)
    ones = jnp.ones((16,), jnp.int32)
    lane0 = lanes == 0
    hi8 = lanes >= 8

    for i in range(HIST // 16):
        hist_a[pl.ds(16 * i, 16)] = zeros
        hist_b[pl.ds(16 * i, 16)] = zeros

    def sb_body(sb, _):
        pltpu.sync_copy(lab_hbm.at[pl.ds(base + sb * SBR, SBR)], lab_buf)

        def one_row(r, hist):
            labs = [lab_buf[r, pl.ds(16 * k, 16)] for k in range(12)]
            tail = lab_buf[r, pl.ds(L - 16, 16)]
            for lv in labs:
                plsc.addupdate_scatter(hist, [lv], ones)
            plsc.addupdate_scatter(hist, [tail], ones, mask=hi8)
            return labs, tail

        def finish_row(r, hist, labs, tail):
            acc = jnp.zeros((16,), jnp.float32)
            for lv in labs:
                occ = plsc.load_gather(hist, [lv])
                acc = acc + 1.0 / occ.astype(jnp.float32)
            tocc = plsc.load_gather(hist, [tail])
            acc = acc + jnp.where(
                hi8, 1.0 / tocc.astype(jnp.float32), zerosf)
            s = jnp.sum(acc)
            inc = (s + 0.25).astype(jnp.int32) + 1
            ridx = jnp.full((16,), sb * SBR + r, jnp.int32)
            plsc.store_scatter(inc_buf, [ridx],
                               jnp.full((16,), inc, jnp.int32), mask=lane0)
            for lv in labs:
                plsc.store_scatter(hist, [lv], zeros)
            plsc.store_scatter(hist, [tail], zeros)

        def row_pair(r2, _):
            ra, rb = 2 * r2, 2 * r2 + 1
            la, ta = one_row(ra, hist_a)
            lb, tb = one_row(rb, hist_b)
            finish_row(ra, hist_a, la, ta)
            finish_row(rb, hist_b, lb, tb)
            return 0

        lax.fori_loop(0, SBR // 2, row_pair, 0)
        return 0

    lax.fori_loop(0, NSB, sb_body, 0)

    tacc = zeros
    for j in range(RPT // 16):
        tacc = tacc + inc_buf[pl.ds(16 * j, 16)]
    total = jnp.sum(tacc)
    tot_buf[...] = jnp.full((16,), total, jnp.int32)
    pltpu.sync_copy(inc_buf, inc_hbm.at[pl.ds(base, RPT)])
    pltpu.sync_copy(tot_buf, tot_hbm.at[wid])


@functools.partial(
    pl.kernel,
    out_type=jax.ShapeDtypeStruct((PB, 128), jnp.float32),
    mesh=_mesh,
    compiler_params=_sc_params,
    scratch_types=[
        pltpu.VMEM((RPT,), jnp.int32),
        pltpu.VMEM((NW, 16), jnp.int32),
        pltpu.VMEM((RPT,), jnp.int32),
        pltpu.VMEM((16, 128), jnp.int32),
        pltpu.VMEM((16, 128), jnp.int32),
        pltpu.VMEM((16, 128), jnp.float32),
        pltpu.VMEM((16, 128), jnp.float32),
        pltpu.VMEM((16, 128), jnp.float32),
        pltpu.SemaphoreType.DMA,
        pltpu.SemaphoreType.DMA,
        pltpu.SemaphoreType.DMA,
        pltpu.SemaphoreType.DMA,
    ],
)
def _sc_gather(inp_hbm, inc_hbm, tot_hbm, cl_hbm, out_hbm, inc_buf, tots,
               offs_buf, cl_a, cl_b, win_a, win_b, val_buf,
               sem_wa, sem_wb, sem_ca, sem_cb):
    wid = lax.axis_index("s") * NC + lax.axis_index("c")
    base = wid * RPT
    pbase = wid * PPT
    pltpu.sync_copy(inc_hbm.at[pl.ds(base, RPT)], inc_buf)
    pltpu.sync_copy(tot_hbm, tots)

    acc = jnp.zeros((16,), jnp.int32)
    for w in range(NW):
        row = tots[w, pl.ds(0, 16)]
        acc = acc + jnp.where(w < wid, row, 0)
    gbase = jnp.max(acc)

    def cums(j, carry):
        v = inc_buf[pl.ds(16 * j, 16)]
        c = plsc.cumsum(v)
        offs_buf[pl.ds(16 * j, 16)] = c - v + carry
        return carry + jnp.max(c)

    lax.fori_loop(0, RPT // 16, cums, gbase)

    def issue(sb, cl_buf, win_buf, sem_w, sem_c):
        pltpu.async_copy(cl_hbm.at[pl.ds(pbase + 16 * sb, 16)], cl_buf,
                         sem_c)

        def row_win(r, _):
            off_v = plsc.load_gather(
                offs_buf, [jnp.full((16,), sb * SBR + r, jnp.int32)])
            off_s = jnp.max(off_v)
            start = pl.multiple_of(off_s - lax.rem(off_s, 8), 8)
            prow = lax.div(r, 2)
            col0 = pl.multiple_of(lax.rem(r, 2) * 64, 64)
            pltpu.async_copy(inp_hbm.at[pl.ds(start, 64)],
                             win_buf.at[prow, pl.ds(col0, 64)], sem_w)
            return 0

        lax.fori_loop(0, SBR, row_win, 0)

    def drain_sel(sb, cl_buf, win_buf, sem_w, sem_c):
        pltpu.make_async_copy(
            cl_hbm.at[pl.ds(pbase + 16 * sb, 16)], cl_buf, sem_c).wait()
        pltpu.make_async_copy(
            out_hbm.at[pl.ds(pbase + 16 * sb, 16)], win_buf, sem_w).wait()

        def row_sel(r, _):
            off_v = plsc.load_gather(
                offs_buf, [jnp.full((16,), sb * SBR + r, jnp.int32)])
            off_lo = lax.rem(off_v, 8)
            prow = lax.div(r, 2)
            col0 = lax.rem(r, 2) * 64
            prow_v = jnp.full((16,), prow, jnp.int32)
            for k in range(CP // 16):
                clk = cl_buf[prow, pl.ds(col0 + 16 * k, 16)]
                v = plsc.load_gather(win_buf, [prow_v, clk + off_lo + col0])
                val_buf[prow, pl.ds(col0 + 16 * k, 16)] = v
            return 0

        lax.fori_loop(0, SBR, row_sel, 0)
        pltpu.sync_copy(val_buf, out_hbm.at[pl.ds(pbase + 16 * sb, 16)])

    issue(0, cl_a, win_a, sem_wa, sem_ca)

    def sb_body(sb, _):
        even = lax.rem(sb, 2) == 0
        more = sb + 1 < NSB

        @pl.when(jnp.logical_and(more, even))
        def _():
            issue(sb + 1, cl_b, win_b, sem_wb, sem_cb)

        @pl.when(jnp.logical_and(more, jnp.logical_not(even)))
        def _():
            issue(sb + 1, cl_a, win_a, sem_wa, sem_ca)

        @pl.when(even)
        def _():
            drain_sel(sb, cl_a, win_a, sem_wa, sem_ca)

        @pl.when(jnp.logical_not(even))
        def _():
            drain_sel(sb, cl_b, win_b, sem_wb, sem_cb)

        return 0

    lax.fori_loop(0, NSB, sb_body, 0)


_TC_GRID = 32
_TC_ROWS = PB // _TC_GRID


def _tc_loss_body(v_ref, out_ref, acc_ref):
    i = pl.program_id(0)

    @pl.when(i == 0)
    def _():
        acc_ref[...] = jnp.zeros((8, 128), jnp.float32)

    x = v_ref[...]
    t = -jnp.log(jax.nn.sigmoid(x) + 1e-08)
    col = lax.broadcasted_iota(jnp.int32, (_TC_ROWS, 128), 1)
    t = jnp.where(lax.rem(col, 64) < C, t, 0.0)
    acc_ref[...] += t.reshape(_TC_ROWS // 8, 8, 128).sum(axis=0)

    @pl.when(i == _TC_GRID - 1)
    def _():
        out_ref[0, 0] = jnp.sum(acc_ref[...]) * (1.0 / (B * C))


_tc_loss = pl.pallas_call(
    _tc_loss_body,
    out_shape=jax.ShapeDtypeStruct((1, 1), jnp.float32),
    grid=(_TC_GRID,),
    in_specs=[pl.BlockSpec((_TC_ROWS, 128), lambda i: (i, 0))],
    out_specs=pl.BlockSpec(memory_space=pltpu.SMEM),
    scratch_shapes=[pltpu.VMEM((8, 128), jnp.float32)],
)


def kernel(input, labels, labels_clicked):
    labels = labels.astype(jnp.int32)
    clicked = labels_clicked.astype(jnp.int32)
    cl_pad = jnp.concatenate(
        [clicked, jnp.zeros((B, CP - C), jnp.int32)], axis=1).reshape(PB, 128)
    inc, tot = _sc_unique_counts(labels)
    vals = _sc_gather(input, inc, tot, cl_pad)
    return _tc_loss(vals)[0, 0]


# speedup vs baseline: 1.2629x; 1.1250x over previous
import functools

import jax
import jax.numpy as jnp
from jax import lax
from jax.experimental import pallas as pl
from jax.experimental.pallas import tpu as pltpu
from jax.experimental.pallas import tpu_sc as plsc

B = 16384
L = 200
C = 50
CP = 64
M = B * 202
NC, NS = 2, 16
NW = NC * NS
RPT = B // NW
SBR = 32
NSB = RPT // SBR
HIST = 1024
PB = B // 2
PPT = PB // NW

_mesh = plsc.VectorSubcoreMesh(core_axis_name="c", subcore_axis_name="s")
_sc_params = pltpu.CompilerParams(needs_layout_passes=False)


@functools.partial(
    pl.kernel,
    out_type=[
        jax.ShapeDtypeStruct((B,), jnp.int32),
        jax.ShapeDtypeStruct((NW, 16), jnp.int32),
    ],
    mesh=_mesh,
    compiler_params=_sc_params,
    scratch_types=[
        pltpu.VMEM((SBR, L), jnp.int32),
        pltpu.VMEM((SBR, L), jnp.int32),
        pltpu.VMEM((HIST,), jnp.int32),
        pltpu.VMEM((HIST,), jnp.int32),
        pltpu.VMEM((RPT,), jnp.int32),
        pltpu.VMEM((16,), jnp.int32),
        pltpu.SemaphoreType.DMA,
        pltpu.SemaphoreType.DMA,
    ],
)
def _sc_unique_counts(lab_hbm, inc_hbm, tot_hbm, lab_a, lab_b, hist_a,
                      hist_b, inc_buf, tot_buf, sem_la, sem_lb):
    wid = lax.axis_index("s") * NC + lax.axis_index("c")
    base = wid * RPT
    lanes = lax.iota(jnp.int32, 16)
    zeros = jnp.zeros((16,), jnp.int32)
    zerosf = jnp.zeros((16,), jnp.float32)
    ones = jnp.ones((16,), jnp.int32)
    lane0 = lanes == 0
    hi8 = lanes >= 8

    for i in range(HIST // 16):
        hist_a[pl.ds(16 * i, 16)] = zeros
        hist_b[pl.ds(16 * i, 16)] = zeros

    def lab_issue(sb, lab_buf, sem_l):
        pltpu.async_copy(lab_hbm.at[pl.ds(base + sb * SBR, SBR)], lab_buf,
                         sem_l)

    lab_issue(0, lab_a, sem_la)

    def sb_body(sb, lab_buf, sem_l):
        pltpu.make_async_copy(
            lab_hbm.at[pl.ds(base + sb * SBR, SBR)], lab_buf, sem_l).wait()

        def one_row(r, hist):
            labs = [lab_buf[r, pl.ds(16 * k, 16)] for k in range(12)]
            tail = lab_buf[r, pl.ds(L - 16, 16)]
            for lv in labs:
                plsc.addupdate_scatter(hist, [lv], ones)
            plsc.addupdate_scatter(hist, [tail], ones, mask=hi8)
            return labs, tail

        def finish_row(r, hist, labs, tail):
            acc = jnp.zeros((16,), jnp.float32)
            for lv in labs:
                occ = plsc.load_gather(hist, [lv])
                acc = acc + 1.0 / occ.astype(jnp.float32)
            tocc = plsc.load_gather(hist, [tail])
            acc = acc + jnp.where(
                hi8, 1.0 / tocc.astype(jnp.float32), zerosf)
            s = jnp.sum(acc)
            inc = (s + 0.25).astype(jnp.int32) + 1
            ridx = jnp.full((16,), sb * SBR + r, jnp.int32)
            plsc.store_scatter(inc_buf, [ridx],
                               jnp.full((16,), inc, jnp.int32), mask=lane0)
            for lv in labs:
                plsc.store_scatter(hist, [lv], zeros)
            plsc.store_scatter(hist, [tail], zeros)

        def row_pair(r2, _):
            ra, rb = 2 * r2, 2 * r2 + 1
            la, ta = one_row(ra, hist_a)
            lb, tb = one_row(rb, hist_b)
            finish_row(ra, hist_a, la, ta)
            finish_row(rb, hist_b, lb, tb)
            return 0

        lax.fori_loop(0, SBR // 2, row_pair, 0)

    def sb_step(sb, _):
        even = lax.rem(sb, 2) == 0
        more = sb + 1 < NSB

        @pl.when(jnp.logical_and(more, even))
        def _():
            lab_issue(sb + 1, lab_b, sem_lb)

        @pl.when(jnp.logical_and(more, jnp.logical_not(even)))
        def _():
            lab_issue(sb + 1, lab_a, sem_la)

        @pl.when(even)
        def _():
            sb_body(sb, lab_a, sem_la)

        @pl.when(jnp.logical_not(even))
        def _():
            sb_body(sb, lab_b, sem_lb)

        return 0

    lax.fori_loop(0, NSB, sb_step, 0)

    tacc = zeros
    for j in range(RPT // 16):
        tacc = tacc + inc_buf[pl.ds(16 * j, 16)]
    total = jnp.sum(tacc)
    tot_buf[...] = jnp.full((16,), total, jnp.int32)
    pltpu.sync_copy(inc_buf, inc_hbm.at[pl.ds(base, RPT)])
    pltpu.sync_copy(tot_buf, tot_hbm.at[wid])


@functools.partial(
    pl.kernel,
    out_type=jax.ShapeDtypeStruct((PB, 128), jnp.float32),
    mesh=_mesh,
    compiler_params=_sc_params,
    scratch_types=[
        pltpu.VMEM((RPT,), jnp.int32),
        pltpu.VMEM((NW, 16), jnp.int32),
        pltpu.VMEM((RPT,), jnp.int32),
        pltpu.VMEM((16, 128), jnp.int32),
        pltpu.VMEM((16, 128), jnp.int32),
        pltpu.VMEM((16, 128), jnp.float32),
        pltpu.VMEM((16, 128), jnp.float32),
        pltpu.VMEM((16, 128), jnp.float32),
        pltpu.SemaphoreType.DMA,
        pltpu.SemaphoreType.DMA,
        pltpu.SemaphoreType.DMA,
        pltpu.SemaphoreType.DMA,
    ],
)
def _sc_gather(inp_hbm, inc_hbm, tot_hbm, cl_hbm, out_hbm, inc_buf, tots,
               offs_buf, cl_a, cl_b, win_a, win_b, val_buf,
               sem_wa, sem_wb, sem_ca, sem_cb):
    wid = lax.axis_index("s") * NC + lax.axis_index("c")
    base = wid * RPT
    pbase = wid * PPT
    pltpu.sync_copy(inc_hbm.at[pl.ds(base, RPT)], inc_buf)
    pltpu.sync_copy(tot_hbm, tots)

    acc = jnp.zeros((16,), jnp.int32)
    for w in range(NW):
        row = tots[w, pl.ds(0, 16)]
        acc = acc + jnp.where(w < wid, row, 0)
    gbase = jnp.max(acc)

    def cums(j, carry):
        v = inc_buf[pl.ds(16 * j, 16)]
        c = plsc.cumsum(v)
        offs_buf[pl.ds(16 * j, 16)] = c - v + carry
        return carry + jnp.max(c)

    lax.fori_loop(0, RPT // 16, cums, gbase)

    def issue(sb, cl_buf, win_buf, sem_w, sem_c):
        pltpu.async_copy(cl_hbm.at[pl.ds(pbase + 16 * sb, 16)], cl_buf,
                         sem_c)

        def row_win(r, _):
            off_v = plsc.load_gather(
                offs_buf, [jnp.full((16,), sb * SBR + r, jnp.int32)])
            off_s = jnp.max(off_v)
            start = pl.multiple_of(off_s - lax.rem(off_s, 8), 8)
            prow = lax.div(r, 2)
            col0 = pl.multiple_of(lax.rem(r, 2) * 64, 64)
            pltpu.async_copy(inp_hbm.at[pl.ds(start, 64)],
                             win_buf.at[prow, pl.ds(col0, 64)], sem_w)
            return 0

        lax.fori_loop(0, SBR, row_win, 0)

    def drain_sel(sb, cl_buf, win_buf, sem_w, sem_c):
        pltpu.make_async_copy(
            cl_hbm.at[pl.ds(pbase + 16 * sb, 16)], cl_buf, sem_c).wait()
        pltpu.make_async_copy(
            out_hbm.at[pl.ds(pbase + 16 * sb, 16)], win_buf, sem_w).wait()

        def row_sel(r, _):
            off_v = plsc.load_gather(
                offs_buf, [jnp.full((16,), sb * SBR + r, jnp.int32)])
            off_lo = lax.rem(off_v, 8)
            prow = lax.div(r, 2)
            col0 = lax.rem(r, 2) * 64
            prow_v = jnp.full((16,), prow, jnp.int32)
            for k in range(CP // 16):
                clk = cl_buf[prow, pl.ds(col0 + 16 * k, 16)]
                v = plsc.load_gather(win_buf, [prow_v, clk + off_lo + col0])
                val_buf[prow, pl.ds(col0 + 16 * k, 16)] = v
            return 0

        lax.fori_loop(0, SBR, row_sel, 0)
        pltpu.sync_copy(val_buf, out_hbm.at[pl.ds(pbase + 16 * sb, 16)])

    issue(0, cl_a, win_a, sem_wa, sem_ca)

    def sb_body(sb, _):
        even = lax.rem(sb, 2) == 0
        more = sb + 1 < NSB

        @pl.when(jnp.logical_and(more, even))
        def _():
            issue(sb + 1, cl_b, win_b, sem_wb, sem_cb)

        @pl.when(jnp.logical_and(more, jnp.logical_not(even)))
        def _():
            issue(sb + 1, cl_a, win_a, sem_wa, sem_ca)

        @pl.when(even)
        def _():
            drain_sel(sb, cl_a, win_a, sem_wa, sem_ca)

        @pl.when(jnp.logical_not(even))
        def _():
            drain_sel(sb, cl_b, win_b, sem_wb, sem_cb)

        return 0

    lax.fori_loop(0, NSB, sb_body, 0)


_TC_GRID = 32
_TC_ROWS = PB // _TC_GRID


def _tc_loss_body(v_ref, out_ref, acc_ref):
    i = pl.program_id(0)

    @pl.when(i == 0)
    def _():
        acc_ref[...] = jnp.zeros((8, 128), jnp.float32)

    x = v_ref[...]
    t = -jnp.log(jax.nn.sigmoid(x) + 1e-08)
    col = lax.broadcasted_iota(jnp.int32, (_TC_ROWS, 128), 1)
    t = jnp.where(lax.rem(col, 64) < C, t, 0.0)
    acc_ref[...] += t.reshape(_TC_ROWS // 8, 8, 128).sum(axis=0)

    @pl.when(i == _TC_GRID - 1)
    def _():
        out_ref[0, 0] = jnp.sum(acc_ref[...]) * (1.0 / (B * C))


_tc_loss = pl.pallas_call(
    _tc_loss_body,
    out_shape=jax.ShapeDtypeStruct((1, 1), jnp.float32),
    grid=(_TC_GRID,),
    in_specs=[pl.BlockSpec((_TC_ROWS, 128), lambda i: (i, 0))],
    out_specs=pl.BlockSpec(memory_space=pltpu.SMEM),
    scratch_shapes=[pltpu.VMEM((8, 128), jnp.float32)],
)


def kernel(input, labels, labels_clicked):
    labels = labels.astype(jnp.int32)
    clicked = labels_clicked.astype(jnp.int32)
    cl_pad = jnp.concatenate(
        [clicked, jnp.zeros((B, CP - C), jnp.int32)], axis=1).reshape(PB, 128)
    inc, tot = _sc_unique_counts(labels)
    vals = _sc_gather(input, inc, tot, cl_pad)
    return _tc_loss(vals)[0, 0]
